# baseline (speedup 1.0000x reference)
import functools

import jax
import jax.numpy as jnp
from jax import lax
from jax.experimental import pallas as pl
from jax.experimental.pallas import tpu as pltpu
from jax.experimental.pallas import tpu_sc as plsc

N = 10000
E = 320000
D_H = 128
D_OUT = 40

NC = 2
NS = 16
NW = NC * NS
CHUNK = 128
NCHT = 160
NCH0 = 80
NCH1 = NCHT - NCH0
EP = NS * NCHT * CHUNK
NPAD = 10240
RPW = NPAD // NS
NBUF = 2
PH = 40

ROW_BLK = 1000
GRID = N // ROW_BLK

_mesh = plsc.VectorSubcoreMesh(core_axis_name="c", subcore_axis_name="s")


def _phases(start, count):
    out = []
    done = 0
    while done < count:
        n = min(PH, count - done)
        out.append((start + done, n))
        done += n
    return out


@functools.partial(
    pl.kernel,
    out_type=jax.ShapeDtypeStruct((NC, NPAD, D_H), jnp.float32),
    mesh=_mesh,
    scratch_types=[
        pltpu.VMEM((PH, CHUNK), jnp.int32),
        pltpu.VMEM((CHUNK, D_H), jnp.float32),
        pltpu.VMEM_SHARED((NPAD, D_H), jnp.float32),
        pltpu.SemaphoreType.DMA,
    ],
)
def _sc_count(dstr_hbm, zeros_hbm, ones_hbm, out_hbm,
              dst_all, ones_v, cnt_sh, sem):
    cid = lax.axis_index("c")
    sid = lax.axis_index("s")
    r0 = sid * RPW
    pltpu.sync_copy(zeros_hbm.at[pl.ds(r0, RPW)], cnt_sh.at[pl.ds(r0, RPW)])
    pltpu.sync_copy(ones_hbm, ones_v)
    plsc.subcore_barrier()

    for cid_val in range(NC):
        @pl.when(cid == cid_val)
        def _():
            start = 0 if cid_val == 0 else NCH0
            count = NCH0 if cid_val == 0 else NCH1
            for (ch0, nch_p) in _phases(start, count):
                pltpu.sync_copy(dstr_hbm.at[sid, pl.ds(ch0, nch_p)],
                                dst_all.at[pl.ds(0, nch_p)])

                @pl.loop(0, nch_p)
                def _(j):
                    pltpu.async_copy(ones_v, cnt_sh.at[dst_all.at[j]], sem,
                                     add=True)

                @pl.loop(0, nch_p)
                def _(j):
                    pltpu.make_async_copy(ones_v, cnt_sh.at[dst_all.at[j]],
                                          sem).wait()

    plsc.subcore_barrier()
    pltpu.sync_copy(cnt_sh.at[pl.ds(r0, RPW)], out_hbm.at[cid, pl.ds(r0, RPW)])


@functools.partial(
    pl.kernel,
    out_type=jax.ShapeDtypeStruct((NC, NPAD, D_H), jnp.float32),
    mesh=_mesh,
    scratch_types=[
        pltpu.VMEM((PH, CHUNK), jnp.int32),
        pltpu.VMEM((PH, CHUNK), jnp.int32),
        pltpu.VMEM((NBUF, CHUNK, D_H), jnp.float32),
        pltpu.VMEM_SHARED((NPAD, D_H), jnp.float32),
        pltpu.SemaphoreType.DMA,
        pltpu.SemaphoreType.DMA,
    ],
)
def _sc_agg(g_hbm, srcr_hbm, dstr_hbm, zeros_hbm, out_hbm,
            src_all, dst_all, rows_v, agg_sh, s0, s1):
    sems = [s0, s1]
    cid = lax.axis_index("c")
    sid = lax.axis_index("s")
    r0 = sid * RPW
    pltpu.sync_copy(zeros_hbm.at[pl.ds(r0, RPW)], agg_sh.at[pl.ds(r0, RPW)])
    plsc.subcore_barrier()

    for cid_val in range(NC):
        @pl.when(cid == cid_val)
        def _():
            start = 0 if cid_val == 0 else NCH0
            count = NCH0 if cid_val == 0 else NCH1
            for (ch0, nch_p) in _phases(start, count):
                pltpu.sync_copy(srcr_hbm.at[sid, pl.ds(ch0, nch_p)],
                                src_all.at[pl.ds(0, nch_p)])
                pltpu.sync_copy(dstr_hbm.at[sid, pl.ds(ch0, nch_p)],
                                dst_all.at[pl.ds(0, nch_p)])

                for b in range(NBUF):
                    pltpu.async_copy(g_hbm.at[src_all.at[b]], rows_v.at[b],
                                     sems[b])

                @pl.loop(0, NBUF * ((nch_p + NBUF - 1) // NBUF), step=NBUF)
                def _(k0):
                    for b in range(NBUF):
                        k = k0 + b

                        @pl.when(k < nch_p)
                        def _():
                            pltpu.make_async_copy(
                                g_hbm.at[src_all.at[k]], rows_v.at[b],
                                sems[b]).wait()
                            pltpu.sync_copy(rows_v.at[b],
                                            agg_sh.at[dst_all.at[k]],
                                            add=True)

                            @pl.when(k + NBUF < nch_p)
                            def _():
                                pltpu.async_copy(
                                    g_hbm.at[src_all.at[k + NBUF]],
                                    rows_v.at[b], sems[b])

    plsc.subcore_barrier()
    pltpu.sync_copy(agg_sh.at[pl.ds(r0, RPW)], out_hbm.at[cid, pl.ds(r0, RPW)])


def _dinv_from_counts(c):
    deg = 1.0 + c[0, :, 0:1] + c[1, :, 0:1]
    return lax.rsqrt(deg)


def _g1_body(x_ref, w_ref, c_ref, o_ref):
    h = lax.dot_general(x_ref[...], w_ref[...], (((1,), (0,)), ((), ())),
                        preferred_element_type=jnp.float32,
                        precision=lax.Precision.HIGHEST)
    o_ref[...] = h * _dinv_from_counts(c_ref[...])


def _tc_g1(x, W, counts):
    return pl.pallas_call(
        _g1_body,
        grid=(GRID,),
        in_specs=[
            pl.BlockSpec((ROW_BLK, D_H), lambda i: (i, 0)),
            pl.BlockSpec((D_H, D_H), lambda i: (0, 0)),
            pl.BlockSpec((NC, ROW_BLK, D_H), lambda i: (0, i, 0)),
        ],
        out_specs=pl.BlockSpec((ROW_BLK, D_H), lambda i: (i, 0)),
        out_shape=jax.ShapeDtypeStruct((N, D_H), jnp.float32),
    )(x, W, counts)


def _mid_body(p_ref, g_ref, c_ref, b_ref, w_ref, o_ref):
    dinv = _dinv_from_counts(c_ref[...])
    s = p_ref[0] + p_ref[1] + g_ref[...]
    a = jnp.maximum(s * dinv + b_ref[...], 0.0)
    h = lax.dot_general(a, w_ref[...], (((1,), (0,)), ((), ())),
                        preferred_element_type=jnp.float32,
                        precision=lax.Precision.HIGHEST)
    o_ref[...] = h * dinv


def _tc_mid(parts, g1, counts, b1, W2):
    return pl.pallas_call(
        _mid_body,
        grid=(GRID,),
        in_specs=[
            pl.BlockSpec((NC, ROW_BLK, D_H), lambda i: (0, i, 0)),
            pl.BlockSpec((ROW_BLK, D_H), lambda i: (i, 0)),
            pl.BlockSpec((NC, ROW_BLK, D_H), lambda i: (0, i, 0)),
            pl.BlockSpec((1, D_H), lambda i: (0, 0)),
            pl.BlockSpec((D_H, D_H), lambda i: (0, 0)),
        ],
        out_specs=pl.BlockSpec((ROW_BLK, D_H), lambda i: (i, 0)),
        out_shape=jax.ShapeDtypeStruct((N, D_H), jnp.float32),
    )(parts, g1, counts, b1.reshape(1, D_H), W2)


def _out_body(p_ref, g_ref, c_ref, b_ref, w_ref, bc_ref, o_ref):
    dinv = _dinv_from_counts(c_ref[...])
    s = p_ref[0] + p_ref[1] + g_ref[...]
    a = jnp.maximum(s * dinv + b_ref[...], 0.0)
    o_ref[...] = lax.dot_general(a, w_ref[...], (((1,), (0,)), ((), ())),
                                 preferred_element_type=jnp.float32,
                                 precision=lax.Precision.HIGHEST) + bc_ref[...]


def _tc_out(parts, g2, counts, b2, Wc, bc):
    return pl.pallas_call(
        _out_body,
        grid=(GRID,),
        in_specs=[
            pl.BlockSpec((NC, ROW_BLK, D_H), lambda i: (0, i, 0)),
            pl.BlockSpec((ROW_BLK, D_H), lambda i: (i, 0)),
            pl.BlockSpec((NC, ROW_BLK, D_H), lambda i: (0, i, 0)),
            pl.BlockSpec((1, D_H), lambda i: (0, 0)),
            pl.BlockSpec((D_H, D_OUT), lambda i: (0, 0)),
            pl.BlockSpec((1, D_OUT), lambda i: (0, 0)),
        ],
        out_specs=pl.BlockSpec((ROW_BLK, D_OUT), lambda i: (i, 0)),
        out_shape=jax.ShapeDtypeStruct((N, D_OUT), jnp.float32),
    )(parts, g2, counts, b2.reshape(1, D_H), Wc, bc.reshape(1, D_OUT))


def kernel(x, edge_index, W1, b1, W2, b2, Wc, bc):
    src = edge_index[0]
    dst = edge_index[1]
    pad_src = jnp.zeros((EP - E,), jnp.int32)
    pad_dst = N + (jnp.arange(EP - E, dtype=jnp.int32) % (NPAD - N))
    srcr = jnp.concatenate([src, pad_src]).reshape(NS, NCHT, CHUNK)
    dstr = jnp.concatenate([dst, pad_dst]).reshape(NS, NCHT, CHUNK)
    zeros128 = jnp.zeros((NPAD, D_H), jnp.float32)
    ones128 = jnp.ones((CHUNK, D_H), jnp.float32)
    counts = _sc_count(dstr, zeros128, ones128)
    g1 = _tc_g1(x, W1, counts)
    p1 = _sc_agg(g1, srcr, dstr, zeros128)
    g2 = _tc_mid(p1, g1, counts, b1, W2)
    p2 = _sc_agg(g2, srcr, dstr, zeros128)
    return _tc_out(p2, g2, counts, b2, Wc, bc)


# speedup vs baseline: 2.8647x; 2.8647x over previous
import functools

import jax
import jax.numpy as jnp
from jax import lax
from jax.experimental import pallas as pl
from jax.experimental.pallas import tpu as pltpu
from jax.experimental.pallas import tpu_sc as plsc

N = 10000
E = 320000
D_H = 128
D_OUT = 40

NC = 2
NS = 16
NW = NC * NS
CHUNK = 128
NCHT = 160
NCH0 = 80
NCH1 = NCHT - NCH0
EP = NS * NCHT * CHUNK
NPAD = 10240
RPW = NPAD // NS
NBUF = 2
PH = 40

ROW_BLK = 1000
GRID = N // ROW_BLK

_mesh = plsc.VectorSubcoreMesh(core_axis_name="c", subcore_axis_name="s")


def _phases(start, count):
    out = []
    done = 0
    while done < count:
        n = min(PH, count - done)
        out.append((start + done, n))
        done += n
    return out


@functools.partial(
    pl.kernel,
    out_type=jax.ShapeDtypeStruct((NC, NPAD, D_H), jnp.float32),
    mesh=_mesh,
    scratch_types=[
        pltpu.VMEM((PH, CHUNK), jnp.int32),
        pltpu.VMEM((CHUNK, D_H), jnp.float32),
        pltpu.VMEM_SHARED((NPAD, D_H), jnp.float32),
        pltpu.SemaphoreType.DMA,
    ],
)
def _sc_count(dstr_hbm, zeros_hbm, ones_hbm, out_hbm,
              dst_all, ones_v, cnt_sh, sem):
    cid = lax.axis_index("c")
    sid = lax.axis_index("s")
    r0 = sid * RPW
    pltpu.sync_copy(zeros_hbm.at[pl.ds(r0, RPW)], cnt_sh.at[pl.ds(r0, RPW)])
    pltpu.sync_copy(ones_hbm, ones_v)
    plsc.subcore_barrier()

    for cid_val in range(NC):
        @pl.when(cid == cid_val)
        def _():
            start = 0 if cid_val == 0 else NCH0
            count = NCH0 if cid_val == 0 else NCH1
            for (ch0, nch_p) in _phases(start, count):
                pltpu.sync_copy(dstr_hbm.at[sid, pl.ds(ch0, nch_p)],
                                dst_all.at[pl.ds(0, nch_p)])

                @pl.loop(0, nch_p)
                def _(j):
                    pltpu.async_copy(ones_v, cnt_sh.at[dst_all.at[j]], sem,
                                     add=True)

                @pl.loop(0, nch_p)
                def _(j):
                    pltpu.make_async_copy(ones_v, cnt_sh.at[dst_all.at[j]],
                                          sem).wait()

    plsc.subcore_barrier()
    pltpu.sync_copy(cnt_sh.at[pl.ds(r0, RPW)], out_hbm.at[cid, pl.ds(r0, RPW)])


@functools.partial(
    pl.kernel,
    out_type=jax.ShapeDtypeStruct((NC, NPAD, D_H), jnp.float32),
    mesh=_mesh,
    scratch_types=[
        pltpu.VMEM((PH, CHUNK), jnp.int32),
        pltpu.VMEM((PH, CHUNK), jnp.int32),
        pltpu.VMEM((NBUF, CHUNK, D_H), jnp.float32),
        pltpu.VMEM_SHARED((NPAD, D_H), jnp.float32),
        pltpu.SemaphoreType.DMA,
        pltpu.SemaphoreType.DMA,
    ],
)
def _sc_agg(g_hbm, srcr_hbm, dstr_hbm, zeros_hbm, out_hbm,
            src_all, dst_all, rows_v, agg_sh, s0, s1):
    sems = [s0, s1]
    cid = lax.axis_index("c")
    sid = lax.axis_index("s")
    r0 = sid * RPW
    pltpu.sync_copy(zeros_hbm.at[pl.ds(r0, RPW)], agg_sh.at[pl.ds(r0, RPW)])
    plsc.subcore_barrier()

    for cid_val in range(NC):
        @pl.when(cid == cid_val)
        def _():
            start = 0 if cid_val == 0 else NCH0
            count = NCH0 if cid_val == 0 else NCH1
            for (ch0, nch_p) in _phases(start, count):
                pltpu.sync_copy(srcr_hbm.at[sid, pl.ds(ch0, nch_p)],
                                src_all.at[pl.ds(0, nch_p)])
                pltpu.sync_copy(dstr_hbm.at[sid, pl.ds(ch0, nch_p)],
                                dst_all.at[pl.ds(0, nch_p)])

                for b in range(NBUF):
                    pltpu.async_copy(g_hbm.at[src_all.at[b]], rows_v.at[b],
                                     sems[b])

                @pl.loop(0, NBUF * ((nch_p + NBUF - 1) // NBUF), step=NBUF)
                def _(k0):
                    for b in range(NBUF):
                        k = k0 + b

                        @pl.when(k < nch_p)
                        def _():
                            pltpu.make_async_copy(
                                g_hbm.at[src_all.at[k]], rows_v.at[b],
                                sems[b]).wait()
                            pltpu.sync_copy(rows_v.at[b],
                                            agg_sh.at[dst_all.at[k]],
                                            add=True)

                            @pl.when(k + NBUF < nch_p)
                            def _():
                                pltpu.async_copy(
                                    g_hbm.at[src_all.at[k + NBUF]],
                                    rows_v.at[b], sems[b])

    plsc.subcore_barrier()
    pltpu.sync_copy(agg_sh.at[pl.ds(r0, RPW)], out_hbm.at[cid, pl.ds(r0, RPW)])


def _dinv_from_counts(c):
    deg = 1.0 + c[0, :, 0:1] + c[1, :, 0:1]
    return lax.rsqrt(deg)


def _g1_body(x_ref, w_ref, c_ref, o_ref):
    h = lax.dot_general(x_ref[...], w_ref[...], (((1,), (0,)), ((), ())),
                        preferred_element_type=jnp.float32,
                        precision=lax.Precision.HIGHEST)
    o_ref[...] = h * _dinv_from_counts(c_ref[...])


def _tc_g1(x, W, counts):
    return pl.pallas_call(
        _g1_body,
        grid=(GRID,),
        in_specs=[
            pl.BlockSpec((ROW_BLK, D_H), lambda i: (i, 0)),
            pl.BlockSpec((D_H, D_H), lambda i: (0, 0)),
            pl.BlockSpec((NC, ROW_BLK, D_H), lambda i: (0, i, 0)),
        ],
        out_specs=pl.BlockSpec((ROW_BLK, D_H), lambda i: (i, 0)),
        out_shape=jax.ShapeDtypeStruct((N, D_H), jnp.float32),
    )(x, W, counts)


def _mid_body(p_ref, g_ref, c_ref, b_ref, w_ref, o_ref):
    dinv = _dinv_from_counts(c_ref[...])
    s = p_ref[0] + p_ref[1] + g_ref[...]
    a = jnp.maximum(s * dinv + b_ref[...], 0.0)
    h = lax.dot_general(a, w_ref[...], (((1,), (0,)), ((), ())),
                        preferred_element_type=jnp.float32,
                        precision=lax.Precision.HIGHEST)
    o_ref[...] = h * dinv


def _tc_mid(parts, g1, counts, b1, W2):
    return pl.pallas_call(
        _mid_body,
        grid=(GRID,),
        in_specs=[
            pl.BlockSpec((NC, ROW_BLK, D_H), lambda i: (0, i, 0)),
            pl.BlockSpec((ROW_BLK, D_H), lambda i: (i, 0)),
            pl.BlockSpec((NC, ROW_BLK, D_H), lambda i: (0, i, 0)),
            pl.BlockSpec((1, D_H), lambda i: (0, 0)),
            pl.BlockSpec((D_H, D_H), lambda i: (0, 0)),
        ],
        out_specs=pl.BlockSpec((ROW_BLK, D_H), lambda i: (i, 0)),
        out_shape=jax.ShapeDtypeStruct((N, D_H), jnp.float32),
    )(parts, g1, counts, b1.reshape(1, D_H), W2)


def _out_body(p_ref, g_ref, c_ref, b_ref, w_ref, bc_ref, o_ref):
    dinv = _dinv_from_counts(c_ref[...])
    s = p_ref[0] + p_ref[1] + g_ref[...]
    a = jnp.maximum(s * dinv + b_ref[...], 0.0)
    o_ref[...] = lax.dot_general(a, w_ref[...], (((1,), (0,)), ((), ())),
                                 preferred_element_type=jnp.float32,
                                 precision=lax.Precision.HIGHEST) + bc_ref[...]


def _tc_out(parts, g2, counts, b2, Wc, bc):
    return pl.pallas_call(
        _out_body,
        grid=(GRID,),
        in_specs=[
            pl.BlockSpec((NC, ROW_BLK, D_H), lambda i: (0, i, 0)),
            pl.BlockSpec((ROW_BLK, D_H), lambda i: (i, 0)),
            pl.BlockSpec((NC, ROW_BLK, D_H), lambda i: (0, i, 0)),
            pl.BlockSpec((1, D_H), lambda i: (0, 0)),
            pl.BlockSpec((D_H, D_OUT), lambda i: (0, 0)),
            pl.BlockSpec((1, D_OUT), lambda i: (0, 0)),
        ],
        out_specs=pl.BlockSpec((ROW_BLK, D_OUT), lambda i: (i, 0)),
        out_shape=jax.ShapeDtypeStruct((N, D_OUT), jnp.float32),
    )(parts, g2, counts, b2.reshape(1, D_H), Wc, bc.reshape(1, D_OUT))


def kernel(x, edge_index, W1, b1, W2, b2, Wc, bc):
    src = edge_index[0]
    dst = edge_index[1]
    pad_src = jnp.arange(EP - E, dtype=jnp.int32) % N
    pad_dst = N + (jnp.arange(EP - E, dtype=jnp.int32) % (NPAD - N))
    srcr = jnp.concatenate([src, pad_src]).reshape(NS, NCHT, CHUNK)
    dstr = jnp.concatenate([dst, pad_dst]).reshape(NS, NCHT, CHUNK)
    zeros128 = jnp.zeros((NPAD, D_H), jnp.float32)
    ones128 = jnp.ones((CHUNK, D_H), jnp.float32)
    counts = _sc_count(dstr, zeros128, ones128)
    g1 = _tc_g1(x, W1, counts)
    p1 = _sc_agg(g1, srcr, dstr, zeros128)
    g2 = _tc_mid(p1, g1, counts, b1, W2)
    p2 = _sc_agg(g2, srcr, dstr, zeros128)
    return _tc_out(p2, g2, counts, b2, Wc, bc)


# speedup vs baseline: 3.4830x; 1.2158x over previous
import dataclasses
import functools

import jax
import jax.numpy as jnp
from jax import lax
from jax.experimental import pallas as pl
from jax.experimental.pallas import tpu as pltpu
from jax.experimental.pallas import tpu_sc as plsc

N = 10000
E = 320000
D_H = 128
D_OUT = 40

NC = 2
NS = 16
NW = NC * NS
CHUNK = 128
NCHT = 160
NCH0 = 80
NCH1 = NCHT - NCH0
EP = NS * NCHT * CHUNK
NPAD = 10240
RPW = NPAD // NS
NBUF = 2
PH = 40
CROWS = NPAD // 128

ROW_BLK = 1024
GRID = NPAD // ROW_BLK

_mesh = plsc.VectorSubcoreMesh(core_axis_name="c", subcore_axis_name="s")

_cp = pltpu.CompilerParams()
if "needs_layout_passes" in pltpu.CompilerParams.__dataclass_fields__:
    _cp = dataclasses.replace(_cp, needs_layout_passes=False)


def _phases(start, count):
    out = []
    done = 0
    while done < count:
        n = min(PH, count - done)
        out.append((start + done, n))
        done += n
    return out


@functools.partial(
    pl.kernel,
    out_type=jax.ShapeDtypeStruct((NC, CROWS, 128), jnp.float32),
    mesh=_mesh,
    scratch_types=[
        pltpu.VMEM((PH, CHUNK), jnp.int32),
        pltpu.VMEM((CROWS, 128), jnp.float32),
        pltpu.VMEM((CROWS,), jnp.int32),
        pltpu.VMEM_SHARED((CROWS, 128), jnp.float32),
    ],
    compiler_params=_cp,
)
def _sc_count(dstr_hbm, zeros_hbm, out_hbm, dst_all, hist_v, idn_v, acc_sh):
    cid = lax.axis_index("c")
    sid = lax.axis_index("s")
    pltpu.sync_copy(zeros_hbm.at[pl.ds(0, CROWS)], hist_v)

    @pl.when(sid == 0)
    def _():
        pltpu.sync_copy(zeros_hbm.at[pl.ds(0, CROWS)], acc_sh)

    for k in range(CROWS // 16):
        idn_v[pl.ds(16 * k, 16)] = lax.iota(jnp.int32, 16) + (16 * k)
    ones16 = jnp.ones((16,), jnp.float32)
    plsc.subcore_barrier()

    for cid_val in range(NC):
        @pl.when(cid == cid_val)
        def _():
            start = 0 if cid_val == 0 else NCH0
            count = NCH0 if cid_val == 0 else NCH1
            for (ch0, nch_p) in _phases(start, count):
                pltpu.sync_copy(dstr_hbm.at[sid, pl.ds(ch0, nch_p)],
                                dst_all.at[pl.ds(0, nch_p)])

                @pl.loop(0, nch_p)
                def _(j):
                    for k in range(CHUNK // 16):
                        idx = dst_all[j, pl.ds(16 * k, 16)]
                        hi = lax.shift_right_logical(idx, 7)
                        lo = lax.bitwise_and(idx, 127)
                        plsc.addupdate_scatter(hist_v, [hi, lo], ones16)

    pltpu.sync_copy(hist_v, acc_sh.at[idn_v], add=True)
    plsc.subcore_barrier()

    @pl.when(sid == 0)
    def _():
        pltpu.sync_copy(acc_sh, out_hbm.at[cid])


@functools.partial(
    pl.kernel,
    out_type=jax.ShapeDtypeStruct((NC, NPAD, D_H), jnp.float32),
    mesh=_mesh,
    scratch_types=[
        pltpu.VMEM((PH, CHUNK), jnp.int32),
        pltpu.VMEM((PH, CHUNK), jnp.int32),
        pltpu.VMEM((NBUF, CHUNK, D_H), jnp.float32),
        pltpu.VMEM_SHARED((NPAD, D_H), jnp.float32),
        pltpu.SemaphoreType.DMA,
        pltpu.SemaphoreType.DMA,
    ],
)
def _sc_agg(g_hbm, srcr_hbm, dstr_hbm, zeros_hbm, out_hbm,
            src_all, dst_all, rows_v, agg_sh, s0, s1):
    sems = [s0, s1]
    cid = lax.axis_index("c")
    sid = lax.axis_index("s")
    r0 = sid * RPW
    pltpu.sync_copy(zeros_hbm.at[pl.ds(r0, RPW)], agg_sh.at[pl.ds(r0, RPW)])
    plsc.subcore_barrier()

    for cid_val in range(NC):
        @pl.when(cid == cid_val)
        def _():
            start = 0 if cid_val == 0 else NCH0
            count = NCH0 if cid_val == 0 else NCH1
            for (ch0, nch_p) in _phases(start, count):
                pltpu.sync_copy(srcr_hbm.at[sid, pl.ds(ch0, nch_p)],
                                src_all.at[pl.ds(0, nch_p)])
                pltpu.sync_copy(dstr_hbm.at[sid, pl.ds(ch0, nch_p)],
                                dst_all.at[pl.ds(0, nch_p)])

                for b in range(NBUF):
                    pltpu.async_copy(g_hbm.at[src_all.at[b]], rows_v.at[b],
                                     sems[b])

                @pl.loop(0, NBUF * ((nch_p + NBUF - 1) // NBUF), step=NBUF)
                def _(k0):
                    for b in range(NBUF):
                        k = k0 + b

                        @pl.when(k < nch_p)
                        def _():
                            pltpu.make_async_copy(
                                g_hbm.at[src_all.at[k]], rows_v.at[b],
                                sems[b]).wait()
                            pltpu.sync_copy(rows_v.at[b],
                                            agg_sh.at[dst_all.at[k]],
                                            add=True)

                            @pl.when(k + NBUF < nch_p)
                            def _():
                                pltpu.async_copy(
                                    g_hbm.at[src_all.at[k + NBUF]],
                                    rows_v.at[b], sems[b])

    plsc.subcore_barrier()
    pltpu.sync_copy(agg_sh.at[pl.ds(r0, RPW)], out_hbm.at[cid, pl.ds(r0, RPW)])


def _dinv_from_counts(c):
    deg = 1.0 + c[0] + c[1]
    return lax.rsqrt(deg)


def _scale_rows(v, dinv8):
    v3 = v.reshape(8, 128, D_H)
    return (v3 * dinv8[:, :, None]).reshape(ROW_BLK, D_H)


def _g1_body(x_ref, w_ref, c_ref, o_ref):
    h = lax.dot_general(x_ref[...], w_ref[...], (((1,), (0,)), ((), ())),
                        preferred_element_type=jnp.float32,
                        precision=lax.Precision.HIGHEST)
    o_ref[...] = _scale_rows(h, _dinv_from_counts(c_ref[...]))


def _tc_g1(x, W, counts):
    return pl.pallas_call(
        _g1_body,
        grid=(GRID,),
        in_specs=[
            pl.BlockSpec((ROW_BLK, D_H), lambda i: (i, 0)),
            pl.BlockSpec((D_H, D_H), lambda i: (0, 0)),
            pl.BlockSpec((NC, 8, 128), lambda i: (0, i, 0)),
        ],
        out_specs=pl.BlockSpec((ROW_BLK, D_H), lambda i: (i, 0)),
        out_shape=jax.ShapeDtypeStruct((NPAD, D_H), jnp.float32),
    )(x, W, counts)


def _mid_body(p_ref, g_ref, c_ref, b_ref, w_ref, o_ref):
    dinv8 = _dinv_from_counts(c_ref[...])
    s = p_ref[0] + p_ref[1] + g_ref[...]
    a = jnp.maximum(_scale_rows(s, dinv8) + b_ref[...], 0.0)
    h = lax.dot_general(a, w_ref[...], (((1,), (0,)), ((), ())),
                        preferred_element_type=jnp.float32,
                        precision=lax.Precision.HIGHEST)
    o_ref[...] = _scale_rows(h, dinv8)


def _tc_mid(parts, g1, counts, b1, W2):
    return pl.pallas_call(
        _mid_body,
        grid=(GRID,),
        in_specs=[
            pl.BlockSpec((NC, ROW_BLK, D_H), lambda i: (0, i, 0)),
            pl.BlockSpec((ROW_BLK, D_H), lambda i: (i, 0)),
            pl.BlockSpec((NC, 8, 128), lambda i: (0, i, 0)),
            pl.BlockSpec((1, D_H), lambda i: (0, 0)),
            pl.BlockSpec((D_H, D_H), lambda i: (0, 0)),
        ],
        out_specs=pl.BlockSpec((ROW_BLK, D_H), lambda i: (i, 0)),
        out_shape=jax.ShapeDtypeStruct((NPAD, D_H), jnp.float32),
    )(parts, g1, counts, b1.reshape(1, D_H), W2)


def _out_body(p_ref, g_ref, c_ref, b_ref, w_ref, bc_ref, o_ref):
    dinv8 = _dinv_from_counts(c_ref[...])
    s = p_ref[0] + p_ref[1] + g_ref[...]
    a = jnp.maximum(_scale_rows(s, dinv8) + b_ref[...], 0.0)
    o_ref[...] = lax.dot_general(a, w_ref[...], (((1,), (0,)), ((), ())),
                                 preferred_element_type=jnp.float32,
                                 precision=lax.Precision.HIGHEST) + bc_ref[...]


def _tc_out(parts, g2, counts, b2, Wc, bc):
    return pl.pallas_call(
        _out_body,
        grid=(GRID,),
        in_specs=[
            pl.BlockSpec((NC, ROW_BLK, D_H), lambda i: (0, i, 0)),
            pl.BlockSpec((ROW_BLK, D_H), lambda i: (i, 0)),
            pl.BlockSpec((NC, 8, 128), lambda i: (0, i, 0)),
            pl.BlockSpec((1, D_H), lambda i: (0, 0)),
            pl.BlockSpec((D_H, D_OUT), lambda i: (0, 0)),
            pl.BlockSpec((1, D_OUT), lambda i: (0, 0)),
        ],
        out_specs=pl.BlockSpec((ROW_BLK, D_OUT), lambda i: (i, 0)),
        out_shape=jax.ShapeDtypeStruct((NPAD, D_OUT), jnp.float32),
    )(parts, g2, counts, b2.reshape(1, D_H), Wc, bc.reshape(1, D_OUT))


def kernel(x, edge_index, W1, b1, W2, b2, Wc, bc):
    src = edge_index[0]
    dst = edge_index[1]
    pad_src = jnp.arange(EP - E, dtype=jnp.int32) % N
    pad_dst = N + (jnp.arange(EP - E, dtype=jnp.int32) % (NPAD - N))
    srcr = jnp.concatenate([src, pad_src]).reshape(NS, NCHT, CHUNK)
    dstr = jnp.concatenate([dst, pad_dst]).reshape(NS, NCHT, CHUNK)
    zeros128 = jnp.zeros((NPAD, D_H), jnp.float32)
    x_pad = jnp.concatenate([x, jnp.zeros((NPAD - N, D_H), jnp.float32)])
    counts = _sc_count(dstr, zeros128)
    g1 = _tc_g1(x_pad, W1, counts)
    p1 = _sc_agg(g1, srcr, dstr, zeros128)
    g2 = _tc_mid(p1, g1, counts, b1, W2)
    p2 = _sc_agg(g2, srcr, dstr, zeros128)
    return _tc_out(p2, g2, counts, b2, Wc, bc)[:N]
